# baseline (speedup 1.0000x reference)
import jax
import jax.numpy as jnp
from jax import lax
from jax.experimental import pallas as pl
from jax.experimental.pallas import tpu as pltpu
from jax.experimental.pallas import tpu_sc as plsc

_NUM_CORES = 2
_NUM_SUBCORES = 16
_NUM_WORKERS = _NUM_CORES * _NUM_SUBCORES
_L = 16

_SG = 10


def _embed_kernel(idx_hbm, table_hbm, out_hbm, idx_v, slist_v, gath_v,
                  stage_v, gsem):
  S = idx_hbm.shape[1]
  BT = out_hbm.shape[2]
  bt_per_w = BT // _NUM_WORKERS
  n_sg = S // _SG
  wid = lax.axis_index("s") * _NUM_CORES + lax.axis_index("c")

  lane = lax.iota(jnp.int32, _L)

  for bt_l in range(bt_per_w):
    bt = wid * bt_per_w + bt_l
    pltpu.sync_copy(idx_hbm.at[pl.ds(bt * 128, 128)], idx_v)

    @pl.loop(0, S)
    def _build(s):
      scol = jnp.broadcast_to(s, (_L,))
      for b16 in range(128 // _L):
        rows = b16 * _L + lane
        vals = plsc.load_gather(idx_v, [rows, scol])
        slist_v[s, pl.ds(b16 * _L, _L)] = vals

    for sg in range(n_sg):
      s0 = sg * _SG
      copies = [
          pltpu.async_copy(
              table_hbm.at[slist_v.at[s0 + j]],
              gath_v.at[j],
              gsem,
          )
          for j in range(_SG)
      ]
      for cp in copies:
        cp.wait()

      @pl.loop(0, _SG)
      def _s(sl):
        svec = jnp.broadcast_to(sl, (_L,))
        @pl.loop(0, 32)
        def _f(f):
          fr = f // 8
          fs = f % 8
          fvec = jnp.broadcast_to(f, (_L,))
          for b16 in range(128 // _L):
            bvec = b16 * _L + lane
            vals = plsc.load_gather(gath_v, [svec, bvec, fvec])
            stage_v[sl, fr, fs, pl.ds(b16 * _L, _L)] = vals

      pltpu.sync_copy(stage_v, out_hbm.at[pl.ds(s0, _SG), :, bt])


def kernel(token_ids, weights):
  B0, S = token_ids.shape
  V, D = weights.shape
  mesh = plsc.VectorSubcoreMesh(core_axis_name="c", subcore_axis_name="s")
  run = pl.kernel(
      _embed_kernel,
      out_type=jax.ShapeDtypeStruct((S, 4, B0 // 128, 8, 128), jnp.float32),
      mesh=mesh,
      scratch_types=[
          pltpu.VMEM((128, S), jnp.int32),
          pltpu.VMEM((S, 128), jnp.int32),
          pltpu.VMEM((_SG, 128, D), jnp.float32),
          pltpu.VMEM((_SG, 4, 8, 128), jnp.float32),
          pltpu.SemaphoreType.DMA,
      ],
      compiler_params=pltpu.CompilerParams(
          use_tc_tiling_on_sc=False, needs_layout_passes=False),
  )
  out5 = run(token_ids.astype(jnp.int32), weights)
  return out5.transpose(2, 4, 0, 1, 3).reshape(B0, S, D)


# speedup vs baseline: 1.4343x; 1.4343x over previous
import jax
import jax.numpy as jnp
from jax import lax
from jax.experimental import pallas as pl
from jax.experimental.pallas import tpu as pltpu
from jax.experimental.pallas import tpu_sc as plsc

_NUM_CORES = 2
_NUM_SUBCORES = 16
_NUM_WORKERS = _NUM_CORES * _NUM_SUBCORES
_L = 16

_SG = 5


def _embed_kernel(idx_hbm, table_hbm, out_hbm, idx_v, slist_v, gath0, gath1,
                  stage_v, gsem0, gsem1, osem):
  S = idx_hbm.shape[1]
  BT = out_hbm.shape[2]
  bt_per_w = BT // _NUM_WORKERS
  n_sg = S // _SG
  wid = lax.axis_index("s") * _NUM_CORES + lax.axis_index("c")

  lane = lax.iota(jnp.int32, _L)
  gaths = (gath0, gath1)
  gsems = (gsem0, gsem1)

  def fire(sg, buf):
    for j in range(_SG):
      pltpu.async_copy(
          table_hbm.at[slist_v.at[sg * _SG + j]],
          gaths[buf].at[pl.ds(j * 128, 128)],
          gsems[buf],
      )

  def drain(buf):
    for j in range(_SG):
      pltpu.make_async_copy(
          table_hbm.at[slist_v.at[j]],
          gaths[buf].at[pl.ds(j * 128, 128)],
          gsems[buf],
      ).wait()

  @pl.loop(0, bt_per_w)
  def _bt(bt_l):
    bt = wid * bt_per_w + bt_l
    pltpu.sync_copy(idx_hbm.at[pl.ds(bt * 128, 128)], idx_v)

    @plsc.parallel_loop(0, S, unroll=2)
    def _build(s):
      scol = jnp.broadcast_to(s, (_L,))
      for b16 in range(128 // _L):
        rows = b16 * _L + lane
        vals = plsc.load_gather(idx_v, [rows, scol])
        slist_v[s, pl.ds(b16 * _L, _L)] = vals

    fire(0, 0)
    for sg in range(n_sg):
      buf = sg % 2
      if sg + 1 < n_sg:
        fire(sg + 1, 1 - buf)
      drain(buf)
      if sg >= 2:
        pltpu.make_async_copy(
            stage_v.at[buf],
            out_hbm.at[pl.ds(0, _SG), :, bt],
            osem,
        ).wait()

      gath = gaths[buf]
      @plsc.parallel_loop(0, _SG)
      def _s(sl):
        @plsc.parallel_loop(0, 32, unroll=4)
        def _f(f):
          fr = f // 8
          fs = f % 8
          fvec = jnp.broadcast_to(f, (_L,))
          for b16 in range(128 // _L):
            rows = sl * 128 + b16 * _L + lane
            vals = plsc.load_gather(gath, [rows, fvec])
            stage_v[buf, sl, fr, fs, pl.ds(b16 * _L, _L)] = vals

      pltpu.async_copy(
          stage_v.at[buf],
          out_hbm.at[pl.ds(sg * _SG, _SG), :, bt],
          osem,
      )
    for _ in range(2):
      pltpu.make_async_copy(
          stage_v.at[0],
          out_hbm.at[pl.ds(0, _SG), :, bt],
          osem,
      ).wait()


def kernel(token_ids, weights):
  B0, S = token_ids.shape
  V, D = weights.shape
  mesh = plsc.VectorSubcoreMesh(core_axis_name="c", subcore_axis_name="s")
  run = pl.kernel(
      _embed_kernel,
      out_type=jax.ShapeDtypeStruct((S, 4, B0 // 128, 8, 128), jnp.float32),
      mesh=mesh,
      scratch_types=[
          pltpu.VMEM((128, S), jnp.int32),
          pltpu.VMEM((S, 128), jnp.int32),
          pltpu.VMEM((_SG * 128, D), jnp.float32),
          pltpu.VMEM((_SG * 128, D), jnp.float32),
          pltpu.VMEM((2, _SG, 4, 8, 128), jnp.float32),
          pltpu.SemaphoreType.DMA,
          pltpu.SemaphoreType.DMA,
          pltpu.SemaphoreType.DMA,
      ],
      compiler_params=pltpu.CompilerParams(
          use_tc_tiling_on_sc=False, needs_layout_passes=False),
  )
  out5 = run(token_ids.astype(jnp.int32), weights)
  return out5.transpose(2, 4, 0, 1, 3).reshape(B0, S, D)
